# baseline (speedup 1.0000x reference)
import functools

import jax
import jax.numpy as jnp
from jax import lax
from jax.experimental import pallas as pl
from jax.experimental.pallas import tpu as pltpu
from jax.experimental.pallas import tpu_sc as plsc

NC = 2
NS = 16
NW = NC * NS
LANES = 128
G = 64


def _round_up(v, m):
    return -(-v // m) * m


def _sc_degree(dst3, ones_blk, zeros_n8, n):
    CH = dst3.shape[1]
    n_pad = _round_up(n, NS * 8)
    ZR = n_pad // NS
    mesh = plsc.VectorSubcoreMesh(
        core_axis_name="c", subcore_axis_name="s", num_cores=NC, num_subcores=NS)

    @functools.partial(
        pl.kernel,
        out_type=jax.ShapeDtypeStruct((NC, n_pad, 8), jnp.float32),
        mesh=mesh,
        compiler_params=pltpu.CompilerParams(use_tc_tiling_on_sc=False),
        scratch_types=[
            pltpu.VMEM((CH, LANES), jnp.int32),
            pltpu.VMEM((LANES, 8), jnp.float32),
            pltpu.VMEM_SHARED((n_pad, 8), jnp.float32),
        ],
    )
    def deg_kernel(dst_hbm, ones_hbm, zeros_hbm, out_hbm, dst_v, ones_v, deg_sh):
        cid = lax.axis_index("c")
        sid = lax.axis_index("s")
        wid = sid * NC + cid
        pltpu.sync_copy(zeros_hbm.at[pl.ds(sid * ZR, ZR)],
                        deg_sh.at[pl.ds(sid * ZR, ZR)])
        pltpu.sync_copy(ones_hbm, ones_v)
        pltpu.sync_copy(dst_hbm.at[wid], dst_v)
        plsc.subcore_barrier()

        @pl.loop(0, CH)
        def _(j):
            pltpu.sync_copy(ones_v, deg_sh.at[dst_v.at[j]], add=True)

        plsc.subcore_barrier()
        pltpu.sync_copy(deg_sh.at[pl.ds(sid * ZR, ZR)],
                        out_hbm.at[cid, pl.ds(sid * ZR, ZR)])

    return deg_kernel(dst3, ones_blk, zeros_n8)


def _sc_scatter_rows(hs_pad, src3, dst3, zeros_nh, n, h):
    CH = src3.shape[1]
    n_pad = _round_up(n, NS * 8)
    ZR = n_pad // NS
    mesh = plsc.VectorSubcoreMesh(
        core_axis_name="c", subcore_axis_name="s", num_cores=NC, num_subcores=NS)

    NBUF = 4
    assert CH % NBUF == 0

    @functools.partial(
        pl.kernel,
        out_type=jax.ShapeDtypeStruct((NC, n_pad, h), jnp.float32),
        mesh=mesh,
        compiler_params=pltpu.CompilerParams(use_tc_tiling_on_sc=False),
        scratch_types=[
            pltpu.VMEM((CH, LANES), jnp.int32),
            pltpu.VMEM((CH, LANES), jnp.int32),
            [pltpu.VMEM((LANES, h), jnp.float32) for _ in range(NBUF)],
            pltpu.VMEM_SHARED((n_pad, h), jnp.float32),
            [pltpu.SemaphoreType.DMA for _ in range(NBUF)],
        ],
    )
    def msg_kernel(hs_hbm, src_hbm, dst_hbm, zeros_hbm, out_hbm,
                   src_v, dst_v, rows_v, acc_sh, sems):
        cid = lax.axis_index("c")
        sid = lax.axis_index("s")
        wid = sid * NC + cid
        pltpu.sync_copy(zeros_hbm.at[pl.ds(sid * ZR, ZR)],
                        acc_sh.at[pl.ds(sid * ZR, ZR)])
        pltpu.sync_copy(src_hbm.at[wid], src_v)
        pltpu.sync_copy(dst_hbm.at[wid], dst_v)
        plsc.subcore_barrier()

        for b in range(NBUF - 1):
            pltpu.async_copy(hs_hbm.at[src_v.at[b]], rows_v[b], sems[b])

        @pl.loop(0, CH, step=NBUF)
        def _(j):
            for b in range(NBUF):
                i = j + b
                nxt = i + NBUF - 1

                @pl.when(nxt < CH)
                def _():
                    pltpu.async_copy(hs_hbm.at[src_v.at[nxt]],
                                     rows_v[(b + NBUF - 1) % NBUF],
                                     sems[(b + NBUF - 1) % NBUF])

                pltpu.make_async_copy(hs_hbm.at[src_v.at[i]],
                                      rows_v[b], sems[b]).wait()
                pltpu.sync_copy(rows_v[b], acc_sh.at[dst_v.at[i]], add=True)

        plsc.subcore_barrier()
        pltpu.sync_copy(acc_sh.at[pl.ds(sid * ZR, ZR)],
                        out_hbm.at[cid, pl.ds(sid * ZR, ZR)])

    return msg_kernel(hs_pad, src3, dst3, zeros_nh)


def _tc_matmul_scale(x, W, degp):
    n, d = x.shape
    h = W.shape[1]
    R = 2000

    def body(x_ref, w_ref, deg_ref, hs_ref, dinv_ref):
        deg = deg_ref[0, :, 0:1] + deg_ref[1, :, 0:1] + 1.0
        dinv = lax.rsqrt(deg)
        hm = jnp.dot(x_ref[...], w_ref[...], preferred_element_type=jnp.float32)
        hs_ref[...] = hm * dinv
        dinv_ref[...] = dinv

    return pl.pallas_call(
        body,
        grid=(n // R,),
        in_specs=[
            pl.BlockSpec((R, d), lambda i: (i, 0)),
            pl.BlockSpec((d, h), lambda i: (0, 0)),
            pl.BlockSpec((NC, R, 8), lambda i: (0, i, 0)),
        ],
        out_specs=[
            pl.BlockSpec((R, h), lambda i: (i, 0)),
            pl.BlockSpec((R, 1), lambda i: (i, 0)),
        ],
        out_shape=[
            jax.ShapeDtypeStruct((n, h), jnp.float32),
            jax.ShapeDtypeStruct((n, 1), jnp.float32),
        ],
    )(x, W, degp)


def _tc_layer(acc, hs, dinv, b, W):
    n, h = hs.shape
    h2 = W.shape[1]
    R = 2000

    def body(acc_ref, hs_ref, dinv_ref, b_ref, w_ref, out_ref):
        s = acc_ref[0] + acc_ref[1] + hs_ref[...]
        h1 = jnp.maximum(s * dinv_ref[...] + b_ref[...], 0.0)
        out_ref[...] = jnp.dot(h1, w_ref[...],
                               preferred_element_type=jnp.float32) * dinv_ref[...]

    return pl.pallas_call(
        body,
        grid=(n // R,),
        in_specs=[
            pl.BlockSpec((NC, R, h), lambda i: (0, i, 0)),
            pl.BlockSpec((R, h), lambda i: (i, 0)),
            pl.BlockSpec((R, 1), lambda i: (i, 0)),
            pl.BlockSpec((1, h), lambda i: (0, 0)),
            pl.BlockSpec((h, h2), lambda i: (0, 0)),
        ],
        out_specs=pl.BlockSpec((R, h2), lambda i: (i, 0)),
        out_shape=jax.ShapeDtypeStruct((n, h2), jnp.float32),
    )(acc, hs, dinv, b.reshape(1, h), W)


def _tc_final(acc, hs, dinv, b, bcol, Wc1, bc1, Wc2, bc2):
    n, h = hs.shape
    c1 = Wc1.shape[1]
    c2 = Wc2.shape[1]
    R = 2000
    steps = n // R

    def body(acc_ref, hs_ref, dinv_ref, b_ref, bat_ref, wc1_ref, bc1_ref,
             wc2_ref, bc2_ref, out_ref, gsum, cnt):
        i = pl.program_id(0)

        @pl.when(i == 0)
        def _():
            gsum[...] = jnp.zeros_like(gsum)
            cnt[...] = jnp.zeros_like(cnt)

        s = acc_ref[0] + acc_ref[1] + hs_ref[...]
        hv = jnp.maximum(s * dinv_ref[...] + b_ref[...], 0.0)
        onehot = (bat_ref[...] ==
                  lax.broadcasted_iota(jnp.int32, (R, G), 1)).astype(jnp.float32)
        gsum[...] += lax.dot_general(onehot, hv, (((0,), (0,)), ((), ())),
                                     preferred_element_type=jnp.float32)
        cnt[...] += lax.dot_general(onehot, jnp.ones((R, 1), jnp.float32),
                                    (((0,), (0,)), ((), ())),
                                    preferred_element_type=jnp.float32)

        @pl.when(i == steps - 1)
        def _():
            g = gsum[...] / jnp.maximum(cnt[...], 1.0)
            z = jnp.maximum(jnp.dot(g, wc1_ref[...],
                                    preferred_element_type=jnp.float32)
                            + bc1_ref[...], 0.0)
            out_ref[...] = jnp.dot(z, wc2_ref[...],
                                   preferred_element_type=jnp.float32) + bc2_ref[...]

    return pl.pallas_call(
        body,
        grid=(steps,),
        in_specs=[
            pl.BlockSpec((NC, R, h), lambda i: (0, i, 0)),
            pl.BlockSpec((R, h), lambda i: (i, 0)),
            pl.BlockSpec((R, 1), lambda i: (i, 0)),
            pl.BlockSpec((1, h), lambda i: (0, 0)),
            pl.BlockSpec((R, 1), lambda i: (i, 0)),
            pl.BlockSpec((h, c1), lambda i: (0, 0)),
            pl.BlockSpec((1, c1), lambda i: (0, 0)),
            pl.BlockSpec((c1, c2), lambda i: (0, 0)),
            pl.BlockSpec((1, c2), lambda i: (0, 0)),
        ],
        out_specs=pl.BlockSpec((G, c2), lambda i: (0, 0)),
        out_shape=jax.ShapeDtypeStruct((G, c2), jnp.float32),
        scratch_shapes=[
            pltpu.VMEM((G, h), jnp.float32),
            pltpu.VMEM((G, 1), jnp.float32),
        ],
    )(acc, hs, dinv, b.reshape(1, h), bcol, Wc1, bc1.reshape(1, c1),
      Wc2, bc2.reshape(1, c2))


def kernel(x, edge_index, batch, W1, b1, W2, b2, Wc1, bc1, Wc2, bc2):
    n, d = x.shape
    h = W1.shape[1]
    e = edge_index.shape[1]
    src, dst = edge_index[0], edge_index[1]

    CH = _round_up(-(-e // (NW * LANES)), 4)
    e_pad = NW * CH * LANES
    fill = jnp.full((e_pad - e,), n, jnp.int32)
    src3 = jnp.concatenate([src, fill]).reshape(NW, CH, LANES)
    dst3 = jnp.concatenate([dst, fill]).reshape(NW, CH, LANES)

    np_ = _round_up(n, NS * 8)
    ones_blk = jnp.ones((LANES, 8), jnp.float32)
    zeros_n8 = jnp.zeros((np_, 8), jnp.float32)
    zeros_nh = jnp.zeros((np_, h), jnp.float32)
    zrows = jnp.zeros((np_ - n, h), jnp.float32)

    degp = _sc_degree(dst3, ones_blk, zeros_n8, n)
    hs1, dinv = _tc_matmul_scale(x, W1, degp)
    acc1 = _sc_scatter_rows(jnp.concatenate([hs1, zrows]), src3, dst3,
                            zeros_nh, n, h)
    hs2 = _tc_layer(acc1, hs1, dinv, b1, W2)
    acc2 = _sc_scatter_rows(jnp.concatenate([hs2, zrows]), src3, dst3,
                            zeros_nh, n, h)
    return _tc_final(acc2, hs2, dinv, b2, batch.reshape(n, 1),
                     Wc1, bc1, Wc2, bc2)


# speedup vs baseline: 2.0120x; 2.0120x over previous
import functools

import jax
import jax.numpy as jnp
from jax import lax
from jax.experimental import pallas as pl
from jax.experimental.pallas import tpu as pltpu
from jax.experimental.pallas import tpu_sc as plsc

NC = 2
NS = 16
NW = NC * NS
LANES = 128
G = 64


def _round_up(v, m):
    return -(-v // m) * m


def _sc_degree(dst3, ones_blk, zeros_n8, n):
    CH = dst3.shape[1]
    n_pad = _round_up(n, NS * 8)
    ZR = n_pad // NS
    mesh = plsc.VectorSubcoreMesh(
        core_axis_name="c", subcore_axis_name="s", num_cores=NC, num_subcores=NS)

    @functools.partial(
        pl.kernel,
        out_type=jax.ShapeDtypeStruct((NC, n_pad, 8), jnp.float32),
        mesh=mesh,
        compiler_params=pltpu.CompilerParams(use_tc_tiling_on_sc=False),
        scratch_types=[
            pltpu.VMEM((CH, LANES), jnp.int32),
            pltpu.VMEM((LANES, 8), jnp.float32),
            pltpu.VMEM_SHARED((n_pad, 8), jnp.float32),
        ],
    )
    def deg_kernel(dst_hbm, ones_hbm, zeros_hbm, out_hbm, dst_v, ones_v, deg_sh):
        cid = lax.axis_index("c")
        sid = lax.axis_index("s")
        wid = sid * NC + cid
        pltpu.sync_copy(zeros_hbm.at[pl.ds(sid * ZR, ZR)],
                        deg_sh.at[pl.ds(sid * ZR, ZR)])
        pltpu.sync_copy(ones_hbm, ones_v)
        pltpu.sync_copy(dst_hbm.at[wid], dst_v)
        plsc.subcore_barrier()

        @pl.loop(0, CH)
        def _(j):
            pltpu.sync_copy(ones_v, deg_sh.at[dst_v.at[j]], add=True)

        plsc.subcore_barrier()
        pltpu.sync_copy(deg_sh.at[pl.ds(sid * ZR, ZR)],
                        out_hbm.at[cid, pl.ds(sid * ZR, ZR)])

    return deg_kernel(dst3, ones_blk, zeros_n8)


def _sc_scatter_rows(hs_pad, src3, dst3, zeros_nh, n, h):
    CH = src3.shape[1]
    n_pad = _round_up(n, NS * 8)
    ZR = n_pad // NS
    mesh = plsc.VectorSubcoreMesh(
        core_axis_name="c", subcore_axis_name="s", num_cores=NC, num_subcores=NS)

    @functools.partial(
        pl.kernel,
        out_type=jax.ShapeDtypeStruct((NC, n_pad, h), jnp.float32),
        mesh=mesh,
        compiler_params=pltpu.CompilerParams(use_tc_tiling_on_sc=False),
        scratch_types=[
            pltpu.VMEM((CH, LANES), jnp.int32),
            pltpu.VMEM((CH, LANES), jnp.int32),
            pltpu.VMEM((LANES, h), jnp.float32),
            pltpu.VMEM_SHARED((n_pad, h), jnp.float32),
            pltpu.VMEM_SHARED((n_pad, h), jnp.float32),
            pltpu.SemaphoreType.DMA,
        ],
    )
    def msg_kernel(hs_hbm, src_hbm, dst_hbm, zeros_hbm, out_hbm,
                   src_v, dst_v, rows_v, hs_sh, acc_sh, sem):
        cid = lax.axis_index("c")
        sid = lax.axis_index("s")
        wid = sid * NC + cid
        pltpu.sync_copy(hs_hbm.at[pl.ds(sid * ZR, ZR)],
                        hs_sh.at[pl.ds(sid * ZR, ZR)])
        pltpu.sync_copy(zeros_hbm.at[pl.ds(sid * ZR, ZR)],
                        acc_sh.at[pl.ds(sid * ZR, ZR)])
        pltpu.sync_copy(src_hbm.at[wid], src_v)
        pltpu.sync_copy(dst_hbm.at[wid], dst_v)
        plsc.subcore_barrier()

        @pl.loop(0, CH)
        def _(j):
            pltpu.async_copy(hs_sh.at[src_v.at[j]], rows_v, sem).wait()
            pltpu.sync_copy(rows_v, acc_sh.at[dst_v.at[j]], add=True)

        plsc.subcore_barrier()
        pltpu.sync_copy(acc_sh.at[pl.ds(sid * ZR, ZR)],
                        out_hbm.at[cid, pl.ds(sid * ZR, ZR)])

    return msg_kernel(hs_pad, src3, dst3, zeros_nh)


def _tc_matmul_scale(x, W, degp):
    n, d = x.shape
    h = W.shape[1]
    R = 2000

    def body(x_ref, w_ref, deg_ref, hs_ref, dinv_ref):
        deg = deg_ref[0, :, 0:1] + deg_ref[1, :, 0:1] + 1.0
        dinv = lax.rsqrt(deg)
        hm = jnp.dot(x_ref[...], w_ref[...], preferred_element_type=jnp.float32)
        hs_ref[...] = hm * dinv
        dinv_ref[...] = dinv

    return pl.pallas_call(
        body,
        grid=(n // R,),
        in_specs=[
            pl.BlockSpec((R, d), lambda i: (i, 0)),
            pl.BlockSpec((d, h), lambda i: (0, 0)),
            pl.BlockSpec((NC, R, 8), lambda i: (0, i, 0)),
        ],
        out_specs=[
            pl.BlockSpec((R, h), lambda i: (i, 0)),
            pl.BlockSpec((R, 1), lambda i: (i, 0)),
        ],
        out_shape=[
            jax.ShapeDtypeStruct((n, h), jnp.float32),
            jax.ShapeDtypeStruct((n, 1), jnp.float32),
        ],
    )(x, W, degp)


def _tc_layer(acc, hs, dinv, b, W):
    n, h = hs.shape
    h2 = W.shape[1]
    R = 2000

    def body(acc_ref, hs_ref, dinv_ref, b_ref, w_ref, out_ref):
        s = acc_ref[0] + acc_ref[1] + hs_ref[...]
        h1 = jnp.maximum(s * dinv_ref[...] + b_ref[...], 0.0)
        out_ref[...] = jnp.dot(h1, w_ref[...],
                               preferred_element_type=jnp.float32) * dinv_ref[...]

    return pl.pallas_call(
        body,
        grid=(n // R,),
        in_specs=[
            pl.BlockSpec((NC, R, h), lambda i: (0, i, 0)),
            pl.BlockSpec((R, h), lambda i: (i, 0)),
            pl.BlockSpec((R, 1), lambda i: (i, 0)),
            pl.BlockSpec((1, h), lambda i: (0, 0)),
            pl.BlockSpec((h, h2), lambda i: (0, 0)),
        ],
        out_specs=pl.BlockSpec((R, h2), lambda i: (i, 0)),
        out_shape=jax.ShapeDtypeStruct((n, h2), jnp.float32),
    )(acc, hs, dinv, b.reshape(1, h), W)


def _tc_final(acc, hs, dinv, b, bcol, Wc1, bc1, Wc2, bc2):
    n, h = hs.shape
    c1 = Wc1.shape[1]
    c2 = Wc2.shape[1]
    R = 2000
    steps = n // R

    def body(acc_ref, hs_ref, dinv_ref, b_ref, bat_ref, wc1_ref, bc1_ref,
             wc2_ref, bc2_ref, out_ref, gsum, cnt):
        i = pl.program_id(0)

        @pl.when(i == 0)
        def _():
            gsum[...] = jnp.zeros_like(gsum)
            cnt[...] = jnp.zeros_like(cnt)

        s = acc_ref[0] + acc_ref[1] + hs_ref[...]
        hv = jnp.maximum(s * dinv_ref[...] + b_ref[...], 0.0)
        onehot = (bat_ref[...] ==
                  lax.broadcasted_iota(jnp.int32, (R, G), 1)).astype(jnp.float32)
        gsum[...] += lax.dot_general(onehot, hv, (((0,), (0,)), ((), ())),
                                     preferred_element_type=jnp.float32)
        cnt[...] += lax.dot_general(onehot, jnp.ones((R, 1), jnp.float32),
                                    (((0,), (0,)), ((), ())),
                                    preferred_element_type=jnp.float32)

        @pl.when(i == steps - 1)
        def _():
            g = gsum[...] / jnp.maximum(cnt[...], 1.0)
            z = jnp.maximum(jnp.dot(g, wc1_ref[...],
                                    preferred_element_type=jnp.float32)
                            + bc1_ref[...], 0.0)
            out_ref[...] = jnp.dot(z, wc2_ref[...],
                                   preferred_element_type=jnp.float32) + bc2_ref[...]

    return pl.pallas_call(
        body,
        grid=(steps,),
        in_specs=[
            pl.BlockSpec((NC, R, h), lambda i: (0, i, 0)),
            pl.BlockSpec((R, h), lambda i: (i, 0)),
            pl.BlockSpec((R, 1), lambda i: (i, 0)),
            pl.BlockSpec((1, h), lambda i: (0, 0)),
            pl.BlockSpec((R, 1), lambda i: (i, 0)),
            pl.BlockSpec((h, c1), lambda i: (0, 0)),
            pl.BlockSpec((1, c1), lambda i: (0, 0)),
            pl.BlockSpec((c1, c2), lambda i: (0, 0)),
            pl.BlockSpec((1, c2), lambda i: (0, 0)),
        ],
        out_specs=pl.BlockSpec((G, c2), lambda i: (0, 0)),
        out_shape=jax.ShapeDtypeStruct((G, c2), jnp.float32),
        scratch_shapes=[
            pltpu.VMEM((G, h), jnp.float32),
            pltpu.VMEM((G, 1), jnp.float32),
        ],
    )(acc, hs, dinv, b.reshape(1, h), bcol, Wc1, bc1.reshape(1, c1),
      Wc2, bc2.reshape(1, c2))


def kernel(x, edge_index, batch, W1, b1, W2, b2, Wc1, bc1, Wc2, bc2):
    n, d = x.shape
    h = W1.shape[1]
    e = edge_index.shape[1]
    src, dst = edge_index[0], edge_index[1]

    CH = _round_up(-(-e // (NW * LANES)), 4)
    e_pad = NW * CH * LANES
    fill = jnp.full((e_pad - e,), n, jnp.int32)
    src3 = jnp.concatenate([src, fill]).reshape(NW, CH, LANES)
    dst3 = jnp.concatenate([dst, fill]).reshape(NW, CH, LANES)

    np_ = _round_up(n, NS * 8)
    ones_blk = jnp.ones((LANES, 8), jnp.float32)
    zeros_n8 = jnp.zeros((np_, 8), jnp.float32)
    zeros_nh = jnp.zeros((np_, h), jnp.float32)
    zrows = jnp.zeros((np_ - n, h), jnp.float32)

    degp = _sc_degree(dst3, ones_blk, zeros_n8, n)
    hs1, dinv = _tc_matmul_scale(x, W1, degp)
    acc1 = _sc_scatter_rows(jnp.concatenate([hs1, zrows]), src3, dst3,
                            zeros_nh, n, h)
    hs2 = _tc_layer(acc1, hs1, dinv, b1, W2)
    acc2 = _sc_scatter_rows(jnp.concatenate([hs2, zrows]), src3, dst3,
                            zeros_nh, n, h)
    return _tc_final(acc2, hs2, dinv, b2, batch.reshape(n, 1),
                     Wc1, bc1, Wc2, bc2)


# speedup vs baseline: 2.4596x; 1.2224x over previous
import functools

import jax
import jax.numpy as jnp
from jax import lax
from jax.experimental import pallas as pl
from jax.experimental.pallas import tpu as pltpu
from jax.experimental.pallas import tpu_sc as plsc

NC = 2
NS = 16
NW = NC * NS
LANES = 128
G = 64


def _round_up(v, m):
    return -(-v // m) * m


def _sc_degree(dst3, ones_blk, zeros_n8, n):
    CH = dst3.shape[1]
    n_pad = _round_up(n, NS * 8)
    ZR = n_pad // NS
    mesh = plsc.VectorSubcoreMesh(
        core_axis_name="c", subcore_axis_name="s", num_cores=NC, num_subcores=NS)

    @functools.partial(
        pl.kernel,
        out_type=jax.ShapeDtypeStruct((NC, n_pad, 8), jnp.float32),
        mesh=mesh,
        compiler_params=pltpu.CompilerParams(use_tc_tiling_on_sc=False),
        scratch_types=[
            pltpu.VMEM((CH, LANES), jnp.int32),
            pltpu.VMEM((LANES, 8), jnp.float32),
            pltpu.VMEM_SHARED((n_pad, 8), jnp.float32),
        ],
    )
    def deg_kernel(dst_hbm, ones_hbm, zeros_hbm, out_hbm, dst_v, ones_v, deg_sh):
        cid = lax.axis_index("c")
        sid = lax.axis_index("s")
        wid = sid * NC + cid
        pltpu.sync_copy(zeros_hbm.at[pl.ds(sid * ZR, ZR)],
                        deg_sh.at[pl.ds(sid * ZR, ZR)])
        pltpu.sync_copy(ones_hbm, ones_v)
        pltpu.sync_copy(dst_hbm.at[wid], dst_v)
        plsc.subcore_barrier()

        @pl.loop(0, CH)
        def _(j):
            pltpu.sync_copy(ones_v, deg_sh.at[dst_v.at[j]], add=True)

        plsc.subcore_barrier()
        pltpu.sync_copy(deg_sh.at[pl.ds(sid * ZR, ZR)],
                        out_hbm.at[cid, pl.ds(sid * ZR, ZR)])

    return deg_kernel(dst3, ones_blk, zeros_n8)


def _sc_scatter_rows(hs_pad, src3, dst3, zeros_nh, n, h):
    CH = dst3.shape[1]
    n_pad = _round_up(n, NS * 8)
    ZR = n_pad // NS
    mesh = plsc.VectorSubcoreMesh(
        core_axis_name="c", subcore_axis_name="s", num_cores=NC, num_subcores=NS)

    @functools.partial(
        pl.kernel,
        out_type=jax.ShapeDtypeStruct((NC, n_pad, h), jnp.float32),
        mesh=mesh,
        compiler_params=pltpu.CompilerParams(use_tc_tiling_on_sc=False),
        scratch_types=[
            pltpu.VMEM((CH + 1, LANES), jnp.int32),
            pltpu.VMEM((CH, LANES), jnp.int32),
            [pltpu.VMEM((LANES, h), jnp.float32) for _ in range(2)],
            pltpu.VMEM_SHARED((n_pad, h), jnp.float32),
            pltpu.VMEM_SHARED((n_pad, h), jnp.float32),
            [pltpu.SemaphoreType.DMA for _ in range(2)],
            [pltpu.SemaphoreType.DMA for _ in range(2)],
        ],
    )
    def msg_kernel(hs_hbm, src_hbm, dst_hbm, zeros_hbm, out_hbm,
                   src_v, dst_v, rows, hs_sh, acc_sh, semg, sems):
        cid = lax.axis_index("c")
        sid = lax.axis_index("s")
        wid = sid * NC + cid
        pltpu.sync_copy(hs_hbm.at[pl.ds(sid * ZR, ZR)],
                        hs_sh.at[pl.ds(sid * ZR, ZR)])
        pltpu.sync_copy(zeros_hbm.at[pl.ds(sid * ZR, ZR)],
                        acc_sh.at[pl.ds(sid * ZR, ZR)])
        pltpu.sync_copy(src_hbm.at[wid], src_v)
        pltpu.sync_copy(dst_hbm.at[wid], dst_v)
        plsc.subcore_barrier()

        def gath(c, b):
            pltpu.async_copy(hs_sh.at[src_v.at[c]], rows[b], semg[b])

        def gath_wait(c, b):
            pltpu.make_async_copy(hs_sh.at[src_v.at[c]], rows[b],
                                  semg[b]).wait()

        def scat(c, b):
            pltpu.async_copy(rows[b], acc_sh.at[dst_v.at[c]], sems[b],
                             add=True)

        def scat_wait(c, b):
            pltpu.make_async_copy(rows[b], acc_sh.at[dst_v.at[c]],
                                  sems[b]).wait()

        gath(0, 0)

        @pl.loop(0, CH, step=2)
        def _(j):
            @pl.when(j > 0)
            def _():
                scat_wait(j - 1, 1)

            gath(j + 1, 1)
            gath_wait(j, 0)
            scat(j, 0)
            scat_wait(j, 0)
            gath(j + 2, 0)
            gath_wait(j + 1, 1)
            scat(j + 1, 1)

        gath_wait(CH, 0)
        scat_wait(CH - 1, 1)
        plsc.subcore_barrier()
        pltpu.sync_copy(acc_sh.at[pl.ds(sid * ZR, ZR)],
                        out_hbm.at[cid, pl.ds(sid * ZR, ZR)])

    return msg_kernel(hs_pad, src3, dst3, zeros_nh)


def _tc_matmul_scale(x, W, degp):
    n, d = x.shape
    h = W.shape[1]
    R = 2000

    def body(x_ref, w_ref, deg_ref, hs_ref, dinv_ref):
        deg = deg_ref[0, :, 0:1] + deg_ref[1, :, 0:1] + 1.0
        dinv = lax.rsqrt(deg)
        hm = jnp.dot(x_ref[...], w_ref[...], preferred_element_type=jnp.float32)
        hs_ref[...] = hm * dinv
        dinv_ref[...] = dinv

    return pl.pallas_call(
        body,
        grid=(n // R,),
        in_specs=[
            pl.BlockSpec((R, d), lambda i: (i, 0)),
            pl.BlockSpec((d, h), lambda i: (0, 0)),
            pl.BlockSpec((NC, R, 8), lambda i: (0, i, 0)),
        ],
        out_specs=[
            pl.BlockSpec((R, h), lambda i: (i, 0)),
            pl.BlockSpec((R, 1), lambda i: (i, 0)),
        ],
        out_shape=[
            jax.ShapeDtypeStruct((n, h), jnp.float32),
            jax.ShapeDtypeStruct((n, 1), jnp.float32),
        ],
    )(x, W, degp)


def _tc_layer(acc, hs, dinv, b, W):
    n, h = hs.shape
    h2 = W.shape[1]
    R = 2000

    def body(acc_ref, hs_ref, dinv_ref, b_ref, w_ref, out_ref):
        s = acc_ref[0] + acc_ref[1] + hs_ref[...]
        h1 = jnp.maximum(s * dinv_ref[...] + b_ref[...], 0.0)
        out_ref[...] = jnp.dot(h1, w_ref[...],
                               preferred_element_type=jnp.float32) * dinv_ref[...]

    return pl.pallas_call(
        body,
        grid=(n // R,),
        in_specs=[
            pl.BlockSpec((NC, R, h), lambda i: (0, i, 0)),
            pl.BlockSpec((R, h), lambda i: (i, 0)),
            pl.BlockSpec((R, 1), lambda i: (i, 0)),
            pl.BlockSpec((1, h), lambda i: (0, 0)),
            pl.BlockSpec((h, h2), lambda i: (0, 0)),
        ],
        out_specs=pl.BlockSpec((R, h2), lambda i: (i, 0)),
        out_shape=jax.ShapeDtypeStruct((n, h2), jnp.float32),
    )(acc, hs, dinv, b.reshape(1, h), W)


def _tc_final(acc, hs, dinv, b, bcol, Wc1, bc1, Wc2, bc2):
    n, h = hs.shape
    c1 = Wc1.shape[1]
    c2 = Wc2.shape[1]
    R = 2000
    steps = n // R

    def body(acc_ref, hs_ref, dinv_ref, b_ref, bat_ref, wc1_ref, bc1_ref,
             wc2_ref, bc2_ref, out_ref, gsum, cnt):
        i = pl.program_id(0)

        @pl.when(i == 0)
        def _():
            gsum[...] = jnp.zeros_like(gsum)
            cnt[...] = jnp.zeros_like(cnt)

        s = acc_ref[0] + acc_ref[1] + hs_ref[...]
        hv = jnp.maximum(s * dinv_ref[...] + b_ref[...], 0.0)
        onehot = (bat_ref[...] ==
                  lax.broadcasted_iota(jnp.int32, (R, G), 1)).astype(jnp.float32)
        gsum[...] += lax.dot_general(onehot, hv, (((0,), (0,)), ((), ())),
                                     preferred_element_type=jnp.float32)
        cnt[...] += lax.dot_general(onehot, jnp.ones((R, 1), jnp.float32),
                                    (((0,), (0,)), ((), ())),
                                    preferred_element_type=jnp.float32)

        @pl.when(i == steps - 1)
        def _():
            g = gsum[...] / jnp.maximum(cnt[...], 1.0)
            z = jnp.maximum(jnp.dot(g, wc1_ref[...],
                                    preferred_element_type=jnp.float32)
                            + bc1_ref[...], 0.0)
            out_ref[...] = jnp.dot(z, wc2_ref[...],
                                   preferred_element_type=jnp.float32) + bc2_ref[...]

    return pl.pallas_call(
        body,
        grid=(steps,),
        in_specs=[
            pl.BlockSpec((NC, R, h), lambda i: (0, i, 0)),
            pl.BlockSpec((R, h), lambda i: (i, 0)),
            pl.BlockSpec((R, 1), lambda i: (i, 0)),
            pl.BlockSpec((1, h), lambda i: (0, 0)),
            pl.BlockSpec((R, 1), lambda i: (i, 0)),
            pl.BlockSpec((h, c1), lambda i: (0, 0)),
            pl.BlockSpec((1, c1), lambda i: (0, 0)),
            pl.BlockSpec((c1, c2), lambda i: (0, 0)),
            pl.BlockSpec((1, c2), lambda i: (0, 0)),
        ],
        out_specs=pl.BlockSpec((G, c2), lambda i: (0, 0)),
        out_shape=jax.ShapeDtypeStruct((G, c2), jnp.float32),
        scratch_shapes=[
            pltpu.VMEM((G, h), jnp.float32),
            pltpu.VMEM((G, 1), jnp.float32),
        ],
    )(acc, hs, dinv, b.reshape(1, h), bcol, Wc1, bc1.reshape(1, c1),
      Wc2, bc2.reshape(1, c2))


def kernel(x, edge_index, batch, W1, b1, W2, b2, Wc1, bc1, Wc2, bc2):
    n, d = x.shape
    h = W1.shape[1]
    e = edge_index.shape[1]
    src, dst = edge_index[0], edge_index[1]

    CH = _round_up(-(-e // (NW * LANES)), 4)
    e_pad = NW * CH * LANES
    fill = jnp.full((e_pad - e,), n, jnp.int32)
    src3 = jnp.concatenate([src, fill]).reshape(NW, CH, LANES)
    dst3 = jnp.concatenate([dst, fill]).reshape(NW, CH, LANES)
    src3 = jnp.concatenate(
        [src3, jnp.full((NW, 1, LANES), n, jnp.int32)], axis=1)

    np_ = _round_up(n, NS * 8)
    ones_blk = jnp.ones((LANES, 8), jnp.float32)
    zeros_n8 = jnp.zeros((np_, 8), jnp.float32)
    zeros_nh = jnp.zeros((np_, h), jnp.float32)
    zrows = jnp.zeros((np_ - n, h), jnp.float32)

    degp = _sc_degree(dst3, ones_blk, zeros_n8, n)
    hs1, dinv = _tc_matmul_scale(x, W1, degp)
    acc1 = _sc_scatter_rows(jnp.concatenate([hs1, zrows]), src3, dst3,
                            zeros_nh, n, h)
    hs2 = _tc_layer(acc1, hs1, dinv, b1, W2)
    acc2 = _sc_scatter_rows(jnp.concatenate([hs2, zrows]), src3, dst3,
                            zeros_nh, n, h)
    return _tc_final(acc2, hs2, dinv, b2, batch.reshape(n, 1),
                     Wc1, bc1, Wc2, bc2)
